# baseline (speedup 1.0000x reference)
import functools

import jax
import jax.numpy as jnp
from jax import lax
from jax.experimental import pallas as pl
from jax.experimental.pallas import tpu as pltpu
from jax.experimental.pallas import tpu_sc as plsc

_NC = 2
_NS = 16
_NW = _NC * _NS

_BATCH = 16384
_D_EMB = 128
_N_CHUNKS = 2
_CHUNK = _BATCH // _N_CHUNKS
_B_PER_W = _CHUNK // _NW


_SUB = _B_PER_W // 2


def _gather_body(chunk_base, u_tbl, a_tbl, uid, aid, u_out, a_out,
                 idx_u, idx_a, rows_0, rows_1, sem_0, sem_1):
    wid = lax.axis_index("s") * _NC + lax.axis_index("c")
    base = wid * _B_PER_W
    pltpu.sync_copy(uid.at[pl.ds(chunk_base + base, _B_PER_W)], idx_u)
    pltpu.sync_copy(aid.at[pl.ds(chunk_base + base, _B_PER_W)], idx_a)
    rows = (rows_0, rows_1)
    sems = (sem_0, sem_1)
    items = [
        (u_tbl, idx_u, u_out, 0),
        (u_tbl, idx_u, u_out, _SUB),
        (a_tbl, idx_a, a_out, 0),
        (a_tbl, idx_a, a_out, _SUB),
    ]
    cps = [None, None, None, None]
    for i, (tbl, idx, out, off) in enumerate(items):
        if i >= 2:
            pt, pi, po, poff = items[i - 2]
            cps[i - 2].wait()
            pltpu.sync_copy(rows[i % 2], po.at[pl.ds(base + poff, _SUB)])
        cps[i] = pltpu.async_copy(
            tbl.at[idx.at[pl.ds(off, _SUB)]], rows[i % 2], sems[i % 2]
        )
    for i in (2, 3):
        t, idx, out, off = items[i]
        cps[i].wait()
        pltpu.sync_copy(rows[i % 2], out.at[pl.ds(base + off, _SUB)])


def _make_sc_gather(chunk_base):
    return functools.partial(
        pl.kernel,
        out_type=(
            jax.ShapeDtypeStruct((_CHUNK, _D_EMB), jnp.float32),
            jax.ShapeDtypeStruct((_CHUNK, _D_EMB), jnp.float32),
        ),
        mesh=plsc.VectorSubcoreMesh(core_axis_name="c", subcore_axis_name="s"),
        scratch_types=[
            pltpu.VMEM((_B_PER_W,), jnp.int32),
            pltpu.VMEM((_B_PER_W,), jnp.int32),
            pltpu.VMEM((_SUB, _D_EMB), jnp.float32),
            pltpu.VMEM((_SUB, _D_EMB), jnp.float32),
            pltpu.SemaphoreType.DMA,
            pltpu.SemaphoreType.DMA,
        ],
    )(functools.partial(_gather_body, chunk_base))


_sc_gathers = [_make_sc_gather(c * _CHUNK) for c in range(_N_CHUNKS)]


def _mlp_body(xu_ref, xa_ref, w1u_ref, w1a_ref, b1_ref, w2_ref, b2_ref, o_ref):
    xu = xu_ref[...].astype(jnp.bfloat16)
    xa = xa_ref[...].astype(jnp.bfloat16)
    h = (
        jnp.dot(xu, w1u_ref[...], preferred_element_type=jnp.float32)
        + jnp.dot(xa, w1a_ref[...], preferred_element_type=jnp.float32)
        + b1_ref[...]
    )
    h = jnp.maximum(h, 0.0)
    o = jnp.sum(h * w2_ref[...], axis=1, keepdims=True) + b2_ref[...]
    o_ref[...] = jax.nn.sigmoid(o)


def _mlp(xu, xa, w1u, w1a, b1, w2row, b2, block_b=2048):
    nb = _CHUNK // block_b
    return pl.pallas_call(
        _mlp_body,
        grid=(nb,),
        in_specs=[
            pl.BlockSpec((block_b, _D_EMB), lambda i: (i, 0)),
            pl.BlockSpec((block_b, _D_EMB), lambda i: (i, 0)),
            pl.BlockSpec((_D_EMB, 1024), lambda i: (0, 0)),
            pl.BlockSpec((_D_EMB, 1024), lambda i: (0, 0)),
            pl.BlockSpec((1, 1024), lambda i: (0, 0)),
            pl.BlockSpec((1, 1024), lambda i: (0, 0)),
            pl.BlockSpec((1, 1), lambda i: (0, 0)),
        ],
        out_specs=pl.BlockSpec((block_b, 1), lambda i: (i, 0)),
        out_shape=jax.ShapeDtypeStruct((_CHUNK, 1), jnp.float32),
        compiler_params=pltpu.CompilerParams(
            dimension_semantics=("arbitrary",),
        ),
    )(xu, xa, w1u, w1a, b1, w2row, b2)


@jax.jit
def kernel(userIds, adGroupIds, userTable, adGroupTable, W1, b1, W2, b2):
    uid = userIds.reshape(_BATCH)
    aid = adGroupIds.reshape(_BATCH)
    w1u = W1[:_D_EMB].astype(jnp.bfloat16)
    w1a = W1[_D_EMB:].astype(jnp.bfloat16)
    b1r = b1.reshape(1, 1024)
    w2row = W2.reshape(1, 1024)
    b2r = b2.reshape(1, 1)
    gathered = [g(userTable, adGroupTable, uid, aid) for g in _sc_gathers]
    outs = [_mlp(xu, xa, w1u, w1a, b1r, w2row, b2r) for xu, xa in gathered]
    return jnp.concatenate(outs, axis=0)


# speedup vs baseline: 1.0156x; 1.0156x over previous
import functools

import jax
import jax.numpy as jnp
from jax import lax
from jax.experimental import pallas as pl
from jax.experimental.pallas import tpu as pltpu
from jax.experimental.pallas import tpu_sc as plsc

_NC = 2
_NS = 16
_NW = _NC * _NS

_BATCH = 16384
_D_EMB = 128
_N_CHUNKS = 2
_CHUNK = _BATCH // _N_CHUNKS
_B_PER_W = _CHUNK // _NW


def _gather_body(chunk_base, u_tbl, a_tbl, uid, aid, u_out, a_out,
                 idx_u, idx_a, rows_u, rows_a, sem_u, sem_a):
    wid = lax.axis_index("s") * _NC + lax.axis_index("c")
    base = wid * _B_PER_W
    pltpu.sync_copy(uid.at[pl.ds(chunk_base + base, _B_PER_W)], idx_u)
    pltpu.sync_copy(aid.at[pl.ds(chunk_base + base, _B_PER_W)], idx_a)
    cp_u = pltpu.async_copy(u_tbl.at[idx_u], rows_u, sem_u)
    cp_a = pltpu.async_copy(a_tbl.at[idx_a], rows_a, sem_a)
    cp_u.wait()
    pltpu.sync_copy(rows_u, u_out.at[pl.ds(base, _B_PER_W)])
    cp_a.wait()
    pltpu.sync_copy(rows_a, a_out.at[pl.ds(base, _B_PER_W)])


def _make_sc_gather(chunk_base):
    return functools.partial(
        pl.kernel,
        out_type=(
            jax.ShapeDtypeStruct((_CHUNK, _D_EMB), jnp.float32),
            jax.ShapeDtypeStruct((_CHUNK, _D_EMB), jnp.float32),
        ),
        mesh=plsc.VectorSubcoreMesh(core_axis_name="c", subcore_axis_name="s"),
        scratch_types=[
            pltpu.VMEM((_B_PER_W,), jnp.int32),
            pltpu.VMEM((_B_PER_W,), jnp.int32),
            pltpu.VMEM((_B_PER_W, _D_EMB), jnp.float32),
            pltpu.VMEM((_B_PER_W, _D_EMB), jnp.float32),
            pltpu.SemaphoreType.DMA,
            pltpu.SemaphoreType.DMA,
        ],
    )(functools.partial(_gather_body, chunk_base))


_sc_gathers = [_make_sc_gather(c * _CHUNK) for c in range(_N_CHUNKS)]


def _mlp_body(xu_ref, xa_ref, w1u_ref, w1a_ref, b1_ref, w2_ref, b2_ref, o_ref):
    xu = xu_ref[...].astype(jnp.bfloat16)
    xa = xa_ref[...].astype(jnp.bfloat16)
    h = (
        jnp.dot(xu, w1u_ref[...], preferred_element_type=jnp.float32)
        + jnp.dot(xa, w1a_ref[...], preferred_element_type=jnp.float32)
        + b1_ref[...]
    )
    h = jnp.maximum(h, 0.0)
    o = jnp.sum(h * w2_ref[...], axis=1, keepdims=True) + b2_ref[...]
    o_ref[...] = jax.nn.sigmoid(o)


def _mlp(xu, xa, w1u, w1a, b1, w2row, b2, block_b=2048):
    nb = _CHUNK // block_b
    return pl.pallas_call(
        _mlp_body,
        grid=(nb,),
        in_specs=[
            pl.BlockSpec((block_b, _D_EMB), lambda i: (i, 0)),
            pl.BlockSpec((block_b, _D_EMB), lambda i: (i, 0)),
            pl.BlockSpec((_D_EMB, 1024), lambda i: (0, 0)),
            pl.BlockSpec((_D_EMB, 1024), lambda i: (0, 0)),
            pl.BlockSpec((1, 1024), lambda i: (0, 0)),
            pl.BlockSpec((1, 1024), lambda i: (0, 0)),
            pl.BlockSpec((1, 1), lambda i: (0, 0)),
        ],
        out_specs=pl.BlockSpec((block_b, 1), lambda i: (i, 0)),
        out_shape=jax.ShapeDtypeStruct((_CHUNK, 1), jnp.float32),
        compiler_params=pltpu.CompilerParams(
            dimension_semantics=("arbitrary",),
        ),
    )(xu, xa, w1u, w1a, b1, w2row, b2)


@jax.jit
def kernel(userIds, adGroupIds, userTable, adGroupTable, W1, b1, W2, b2):
    uid = userIds.reshape(_BATCH)
    aid = adGroupIds.reshape(_BATCH)
    w1u = W1[:_D_EMB].astype(jnp.bfloat16)
    w1a = W1[_D_EMB:].astype(jnp.bfloat16)
    b1r = b1.reshape(1, 1024)
    w2row = W2.reshape(1, 1024)
    b2r = b2.reshape(1, 1)
    gathered = [g(userTable, adGroupTable, uid, aid) for g in _sc_gathers]
    outs = [_mlp(xu, xa, w1u, w1a, b1r, w2row, b2r) for xu, xa in gathered]
    return jnp.concatenate(outs, axis=0)
